# Initial kernel scaffold; baseline (speedup 1.0000x reference)
import functools

import jax
import jax.numpy as jnp
from jax import lax
from jax.experimental import pallas as pl

N = 4096
S = 128


def _build_wt_body(rows_ref, vals_ref, wt_ref):
    B = rows_ref.shape[0]
    col = lax.broadcasted_iota(jnp.int32, (B, N), 1)

    def step(s, acc):
        idx = rows_ref[:, pl.ds(s, 1)]
        v = vals_ref[:, pl.ds(s, 1)]
        return acc + jnp.where(col == idx, v, 0.0)

    wt_ref[:, :] = lax.fori_loop(0, S, step, jnp.zeros((B, N), jnp.float32))


def _matmul_body(wt_ref, x_ref, bias_ref, out_ref):
    acc = lax.dot_general(
        wt_ref[:, :], x_ref[:, :],
        dimension_numbers=(((0,), (0,)), ((), ())),
        preferred_element_type=jnp.float32,
    )
    out_ref[:, :] = acc + bias_ref[:, :]


@jax.jit
def kernel(x, rows, cols, vals, bias):
    del cols
    rows2 = rows.reshape(N, S)
    vals2 = vals.reshape(N, S)

    B = 512
    wt = pl.pallas_call(
        _build_wt_body,
        grid=(N // B,),
        in_specs=[
            pl.BlockSpec((B, S), lambda i: (i, 0)),
            pl.BlockSpec((B, S), lambda i: (i, 0)),
        ],
        out_specs=pl.BlockSpec((B, N), lambda i: (i, 0)),
        out_shape=jax.ShapeDtypeStruct((N, N), jnp.float32),
    )(rows2, vals2)

    bm = 512
    bn = 512
    out = pl.pallas_call(
        _matmul_body,
        grid=(N // bm, N // bn),
        in_specs=[
            pl.BlockSpec((N, bm), lambda i, j: (0, i)),
            pl.BlockSpec((N, bn), lambda i, j: (0, j)),
            pl.BlockSpec((1, bn), lambda i, j: (0, j)),
        ],
        out_specs=pl.BlockSpec((bm, bn), lambda i, j: (i, j)),
        out_shape=jax.ShapeDtypeStruct((N, N), jnp.float32),
    )(wt, x, bias.reshape(1, N))
    return out


# speedup vs baseline: 2.2856x; 2.2856x over previous
import functools

import jax
import jax.numpy as jnp
from jax import lax
from jax.experimental import pallas as pl
from jax.experimental.pallas import tpu as pltpu
from jax.experimental.pallas import tpu_sc as plsc

N = 4096
S = 128

NC = 2
NS = 16
NW = NC * NS
CPW = N // NW
CG = 16
KPW = CPW * S

_sc_mesh = plsc.VectorSubcoreMesh(core_axis_name="c", subcore_axis_name="s")


@functools.partial(
    pl.kernel,
    mesh=_sc_mesh,
    out_type=jax.ShapeDtypeStruct((N * N,), jnp.float32),
    scratch_types=[
        pltpu.VMEM((KPW,), jnp.int32),
        pltpu.VMEM((KPW,), jnp.float32),
        pltpu.VMEM((CG * N,), jnp.float32),
    ],
    compiler_params=pltpu.CompilerParams(needs_layout_passes=False),
)
def _build_wt_sc(rows_hbm, vals_hbm, wt_hbm, idx_v, val_v, wbuf):
    wid = lax.axis_index("s") * NC + lax.axis_index("c")
    base_k = wid * KPW
    pltpu.sync_copy(rows_hbm.at[pl.ds(base_k, KPW)], idx_v)
    pltpu.sync_copy(vals_hbm.at[pl.ds(base_k, KPW)], val_v)

    zero16 = jnp.zeros((16,), jnp.float32)

    def zbody(i, carry):
        wbuf[pl.ds(i * 16, 16)] = zero16
        return carry

    lax.fori_loop(0, CG * N // 16, zbody, 0)

    def chunk(g, carry):
        koff = g * (CG * S)
        for j in range(CG):
            for v8 in range(S // 16):
                off = koff + j * S + v8 * 16
                idx = idx_v[pl.ds(off, 16)] + j * N
                val = val_v[pl.ds(off, 16)]
                plsc.addupdate_scatter(wbuf, [idx], val)
        dst = wt_hbm.at[pl.ds(wid * (CPW * N) + g * (CG * N), CG * N)]
        pltpu.sync_copy(wbuf, dst)
        for j in range(CG):
            for v8 in range(S // 16):
                off = koff + j * S + v8 * 16
                idx = idx_v[pl.ds(off, 16)] + j * N
                plsc.store_scatter(wbuf, [idx], zero16)
        return carry

    lax.fori_loop(0, CPW // CG, chunk, 0)


def _matmul_body(wt_ref, x_ref, bias_ref, out_ref):
    acc = lax.dot_general(
        wt_ref[:, :], x_ref[:, :],
        dimension_numbers=(((0,), (0,)), ((), ())),
        preferred_element_type=jnp.float32,
    )
    out_ref[:, :] = acc + bias_ref[:, :]


@jax.jit
def kernel(x, rows, cols, vals, bias):
    del cols

    wt = _build_wt_sc(rows, vals).reshape(N, N)

    bm = 256
    bn = 256
    out = pl.pallas_call(
        _matmul_body,
        grid=(N // bm, N // bn),
        in_specs=[
            pl.BlockSpec((N, bm), lambda i, j: (0, i)),
            pl.BlockSpec((N, bn), lambda i, j: (0, j)),
            pl.BlockSpec((1, bn), lambda i, j: (0, j)),
        ],
        out_specs=pl.BlockSpec((bm, bn), lambda i, j: (i, j)),
        out_shape=jax.ShapeDtypeStruct((N, N), jnp.float32),
    )(wt, x, bias.reshape(1, N))
    return out


# speedup vs baseline: 4.3317x; 1.8952x over previous
import functools

import jax
import jax.numpy as jnp
from jax import lax
from jax.experimental import pallas as pl
from jax.experimental.pallas import tpu as pltpu
from jax.experimental.pallas import tpu_sc as plsc

N = 4096
S = 128

NC = 2
NS = 16
NW = NC * NS
CPW = N // NW
CG = 16
KPW = CPW * S

_sc_mesh = plsc.VectorSubcoreMesh(core_axis_name="c", subcore_axis_name="s")


@functools.partial(
    pl.kernel,
    mesh=_sc_mesh,
    out_type=jax.ShapeDtypeStruct((N * N,), jnp.float32),
    scratch_types=[
        pltpu.VMEM((KPW,), jnp.int32),
        pltpu.VMEM((KPW,), jnp.float32),
        pltpu.VMEM((CG * N,), jnp.float32),
    ],
    compiler_params=pltpu.CompilerParams(needs_layout_passes=False),
)
def _build_wt_sc(rows_hbm, vals_hbm, wt_hbm, idx_v, val_v, wbuf):
    wid = lax.axis_index("s") * NC + lax.axis_index("c")
    base_k = wid * KPW
    pltpu.sync_copy(rows_hbm.at[pl.ds(base_k, KPW)], idx_v)
    pltpu.sync_copy(vals_hbm.at[pl.ds(base_k, KPW)], val_v)

    zero16 = jnp.zeros((16,), jnp.float32)

    def zbody(i, carry):
        wbuf[pl.ds(i * 16, 16)] = zero16
        return carry

    lax.fori_loop(0, CG * N // 16, zbody, 0)

    def chunk(g, carry):
        koff = g * (CG * S)
        for j in range(CG):
            for v8 in range(S // 16):
                off = koff + j * S + v8 * 16
                idx = idx_v[pl.ds(off, 16)] + j * N
                val = val_v[pl.ds(off, 16)]
                plsc.addupdate_scatter(wbuf, [idx], val)
        dst = wt_hbm.at[pl.ds(wid * (CPW * N) + g * (CG * N), CG * N)]
        pltpu.sync_copy(wbuf, dst)
        for j in range(CG):
            for v8 in range(S // 16):
                off = koff + j * S + v8 * 16
                idx = idx_v[pl.ds(off, 16)] + j * N
                plsc.store_scatter(wbuf, [idx], zero16)
        return carry

    lax.fori_loop(0, CPW // CG, chunk, 0)


def _matmul_body(wt_ref, x_ref, bias_ref, out_ref):
    acc = lax.dot_general(
        wt_ref[:, :], x_ref[:, :],
        dimension_numbers=(((0,), (0,)), ((), ())),
        preferred_element_type=jnp.float32,
    )
    out_ref[:, :] = acc + bias_ref[:, :]


@jax.jit
def kernel(x, rows, cols, vals, bias):
    del cols

    wt = _build_wt_sc(rows, vals).reshape(N, N).astype(jnp.bfloat16)
    x = x.astype(jnp.bfloat16)

    bm = 1024
    bn = 1024
    out = pl.pallas_call(
        _matmul_body,
        grid=(N // bm, N // bn),
        in_specs=[
            pl.BlockSpec((N, bm), lambda i, j: (0, i)),
            pl.BlockSpec((N, bn), lambda i, j: (0, j)),
            pl.BlockSpec((1, bn), lambda i, j: (0, j)),
        ],
        out_specs=pl.BlockSpec((bm, bn), lambda i, j: (i, j)),
        out_shape=jax.ShapeDtypeStruct((N, N), jnp.float32),
    )(wt, x, bias.reshape(1, N))
    return out
